# Initial kernel scaffold; baseline (speedup 1.0000x reference)
import jax
import jax.numpy as jnp
from jax.experimental import pallas as pl


def kernel(x, y, W_lookup):
    raise NotImplementedError("write your pallas kernel here")


# speedup vs baseline: 1.5264x; 1.5264x over previous
import functools

import jax
import jax.numpy as jnp
from jax import lax
from jax.experimental import pallas as pl
from jax.experimental.pallas import tpu as pltpu
from jax.experimental.pallas import tpu_sc as plsc

D = 128
LANES = 16
NW = 32
CH = 256


def _lookup_add(x2d, yflat, w):
    n = x2d.shape[0]
    n_w = n // NW
    n_chunks = n_w // CH
    mesh = plsc.VectorSubcoreMesh(core_axis_name="c", subcore_axis_name="s")

    @functools.partial(
        pl.kernel,
        mesh=mesh,
        out_type=jax.ShapeDtypeStruct((n, D), jnp.float32),
        scratch_types=[
            pltpu.VMEM((CH,), jnp.int32),
            pltpu.VMEM((CH, D), jnp.float32),
            pltpu.VMEM((CH, D), jnp.float32),
            pltpu.SemaphoreType.DMA,
        ],
    )
    def k(x_hbm, y_hbm, w_hbm, out_hbm, idx_v, rows_v, xbuf_v, sem):
        wid = lax.axis_index("s") * 2 + lax.axis_index("c")
        w_base = wid * n_w

        def chunk_body(c, carry):
            base = w_base + c * CH
            pltpu.sync_copy(y_hbm.at[pl.ds(base, CH)], idx_v)
            gather = pltpu.async_copy(w_hbm.at[idx_v], rows_v, sem)
            pltpu.sync_copy(x_hbm.at[pl.ds(base, CH)], xbuf_v)
            gather.wait()

            def row_body(j, rcarry):
                for t in range(D // LANES):
                    sl = pl.ds(t * LANES, LANES)
                    plsc.addupdate(xbuf_v.at[j, sl], rows_v[j, sl])
                return rcarry

            lax.fori_loop(0, CH, row_body, 0)
            pltpu.sync_copy(xbuf_v, out_hbm.at[pl.ds(base, CH)])
            return carry

        lax.fori_loop(0, n_chunks, chunk_body, 0)

    return k(x2d, yflat, w)


def kernel(x, y, W_lookup):
    b, h, d = x.shape
    x2d = x.reshape(b * h, d)
    yflat = y.reshape(b * h).astype(jnp.int32)
    out = _lookup_add(x2d, yflat, W_lookup)
    return out.reshape(b, h, d)


# speedup vs baseline: 1.7613x; 1.1539x over previous
import functools

import jax
import jax.numpy as jnp
from jax import lax
from jax.experimental import pallas as pl
from jax.experimental.pallas import tpu as pltpu
from jax.experimental.pallas import tpu_sc as plsc

D = 128
LANES = 16
NW = 32
CH = 128
NR = 2
NX = 5


def _lookup_add(x2d, yflat, w):
    n = x2d.shape[0]
    n_w = n // NW
    n_chunks = n_w // CH
    mesh = plsc.VectorSubcoreMesh(core_axis_name="c", subcore_axis_name="s")

    scratch = (
        [pltpu.VMEM((n_w,), jnp.int32)]
        + [pltpu.VMEM((CH, D), jnp.float32) for _ in range(NR + NX)]
        + [pltpu.SemaphoreType.DMA for _ in range(NR + 2 * NX)]
    )

    @functools.partial(
        pl.kernel,
        mesh=mesh,
        out_type=jax.ShapeDtypeStruct((n, D), jnp.float32),
        scratch_types=scratch,
    )
    def k(x_hbm, y_hbm, w_hbm, out_hbm, idx_all, *bufs_and_sems):
        rows = bufs_and_sems[:NR]
        xb = bufs_and_sems[NR:NR + NX]
        gsem = bufs_and_sems[NR + NX:2 * NR + NX]
        xsem = bufs_and_sems[2 * NR + NX:2 * NR + 2 * NX]
        osem = bufs_and_sems[2 * NR + 2 * NX:]

        wid = lax.axis_index("s") * 2 + lax.axis_index("c")
        w_base = wid * n_w

        pltpu.sync_copy(y_hbm.at[pl.ds(w_base, n_w)], idx_all)

        h_out = [None] * NX

        def issue_gather(c):
            return pltpu.async_copy(
                w_hbm.at[idx_all.at[pl.ds(c * CH, CH)]], rows[c % NR], gsem[c % NR]
            )

        def issue_x(c):
            return pltpu.async_copy(
                x_hbm.at[pl.ds(w_base + c * CH, CH)], xb[c % NX], xsem[c % NX]
            )

        h_g = issue_gather(0)
        h_x = issue_x(0)

        for c in range(n_chunks):
            nh_g = nh_x = None
            if c + 1 < n_chunks:
                nh_g = issue_gather(c + 1)
                if h_out[(c + 1) % NX] is not None:
                    h_out[(c + 1) % NX].wait()
                nh_x = issue_x(c + 1)
            h_g.wait()
            h_x.wait()

            rows_c = rows[c % NR]
            xb_c = xb[c % NX]

            @plsc.parallel_loop(0, CH, 1, unroll=2)
            def add_body(i):
                for t in range(D // LANES):
                    sl = pl.ds(t * LANES, LANES)
                    plsc.addupdate(xb_c.at[i, sl], rows_c[i, sl])

            h_out[c % NX] = pltpu.async_copy(
                xb_c, out_hbm.at[pl.ds(w_base + c * CH, CH)], osem[c % NX]
            )
            h_g, h_x = nh_g, nh_x

        for h in h_out:
            if h is not None:
                h.wait()

    return k(x2d, yflat, w)


def kernel(x, y, W_lookup):
    b, h, d = x.shape
    x2d = x.reshape(b * h, d)
    yflat = y.reshape(b * h).astype(jnp.int32)
    out = _lookup_add(x2d, yflat, W_lookup)
    return out.reshape(b, h, d)


# speedup vs baseline: 3.2365x; 1.8376x over previous
import functools

import jax
import jax.numpy as jnp
from jax import lax
from jax.experimental import pallas as pl
from jax.experimental.pallas import tpu as pltpu
from jax.experimental.pallas import tpu_sc as plsc

D = 128
LANES = 16
NW = 32
NB = 4
NSLOT = 2


def _lookup_add(x, yflat, w):
    bsz, hist, _ = x.shape
    b_w = bsz // NW
    n_chunks = b_w // NB
    rows_per_chunk = NB * hist
    n_w = b_w * hist
    mesh = plsc.VectorSubcoreMesh(core_axis_name="c", subcore_axis_name="s")

    scratch = (
        [pltpu.VMEM((n_w,), jnp.int32)]
        + [pltpu.VMEM((rows_per_chunk, D), jnp.float32) for _ in range(NSLOT)]
        + [pltpu.VMEM((NB, hist, D), jnp.float32) for _ in range(NSLOT)]
        + [pltpu.SemaphoreType.DMA for _ in range(3 * NSLOT)]
    )

    @functools.partial(
        pl.kernel,
        mesh=mesh,
        out_type=jax.ShapeDtypeStruct(x.shape, jnp.float32),
        scratch_types=scratch,
    )
    def k(x_hbm, y_hbm, w_hbm, out_hbm, idx_all, *bufs_and_sems):
        rows = bufs_and_sems[:NSLOT]
        xb = bufs_and_sems[NSLOT:2 * NSLOT]
        gsem = bufs_and_sems[2 * NSLOT:3 * NSLOT]
        xsem = bufs_and_sems[3 * NSLOT:4 * NSLOT]
        osem = bufs_and_sems[4 * NSLOT:]

        wid = lax.axis_index("s") * 2 + lax.axis_index("c")
        b_base = wid * b_w

        pltpu.sync_copy(y_hbm.at[pl.ds(wid * n_w, n_w)], idx_all)

        h_out = [None] * NSLOT

        def issue_gather(c):
            s = c % NSLOT
            return pltpu.async_copy(
                w_hbm.at[idx_all.at[pl.ds(c * rows_per_chunk, rows_per_chunk)]],
                rows[s],
                gsem[s],
            )

        def issue_x(c):
            s = c % NSLOT
            return pltpu.async_copy(
                x_hbm.at[pl.ds(b_base + c * NB, NB)], xb[s], xsem[s]
            )

        h_g = issue_gather(0)
        h_x = issue_x(0)

        for c in range(n_chunks):
            s = c % NSLOT
            nh_g = nh_x = None
            if c + 1 < n_chunks:
                nh_g = issue_gather(c + 1)
                if h_out[(c + 1) % NSLOT] is not None:
                    h_out[(c + 1) % NSLOT].wait()
                nh_x = issue_x(c + 1)
            h_g.wait()
            h_x.wait()

            rows_c = rows[s]
            xb_c = xb[s]

            @plsc.parallel_loop(0, hist, 1)
            def add_body(ih):
                for ib in range(NB):
                    for t in range(D // LANES):
                        sl = pl.ds(t * LANES, LANES)
                        plsc.addupdate(
                            xb_c.at[ib, ih, sl], rows_c[ib * hist + ih, sl]
                        )

            h_out[s] = pltpu.async_copy(
                xb_c, out_hbm.at[pl.ds(b_base + c * NB, NB)], osem[s]
            )
            h_g, h_x = nh_g, nh_x

        for h in h_out:
            if h is not None:
                h.wait()

    return k(x, yflat, w)


def kernel(x, y, W_lookup):
    b, h, d = x.shape
    yflat = y.reshape(b * h).astype(jnp.int32)
    return _lookup_add(x, yflat, W_lookup)


# speedup vs baseline: 6.6538x; 2.0558x over previous
import functools

import jax
import jax.numpy as jnp
from jax import lax
from jax.experimental import pallas as pl
from jax.experimental.pallas import tpu as pltpu
from jax.experimental.pallas import tpu_sc as plsc

D = 128
LANES = 16
NW = 32
CH = 200
NSLOT = 2


def _lookup_add(x2, yflat, w):
    n = x2.shape[0]
    n_w = n // NW
    n_chunks = n_w // CH
    mesh = plsc.VectorSubcoreMesh(core_axis_name="c", subcore_axis_name="s")

    scratch = (
        [pltpu.VMEM((n_w,), jnp.int32)]
        + [pltpu.VMEM((CH, D), jnp.float32) for _ in range(NSLOT)]
        + [pltpu.VMEM((CH, D), jnp.float32) for _ in range(NSLOT)]
        + [pltpu.SemaphoreType.DMA for _ in range(3 * NSLOT)]
    )

    @functools.partial(
        pl.kernel,
        mesh=mesh,
        out_type=jax.ShapeDtypeStruct(x2.shape, jnp.float32),
        scratch_types=scratch,
    )
    def k(x_hbm, y_hbm, w_hbm, out_hbm, idx_all, *bufs_and_sems):
        rows = bufs_and_sems[:NSLOT]
        xb = bufs_and_sems[NSLOT:2 * NSLOT]
        gsem = bufs_and_sems[2 * NSLOT:3 * NSLOT]
        xsem = bufs_and_sems[3 * NSLOT:4 * NSLOT]
        osem = bufs_and_sems[4 * NSLOT:]

        wid = lax.axis_index("s") * 2 + lax.axis_index("c")
        r_base = wid * n_w

        pltpu.sync_copy(y_hbm.at[pl.ds(r_base, n_w)], idx_all)

        h_out = [None] * NSLOT

        def issue_gather(c):
            s = c % NSLOT
            return pltpu.async_copy(
                w_hbm.at[idx_all.at[pl.ds(c * CH, CH)]],
                rows[s],
                gsem[s],
            )

        def issue_x(c):
            s = c % NSLOT
            return pltpu.async_copy(
                x_hbm.at[pl.ds(r_base + c * CH, CH)], xb[s], xsem[s]
            )

        h_g = issue_gather(0)
        h_x = issue_x(0)

        for c in range(n_chunks):
            s = c % NSLOT
            nh_g = nh_x = None
            if c + 1 < n_chunks:
                nh_g = issue_gather(c + 1)
                if h_out[(c + 1) % NSLOT] is not None:
                    h_out[(c + 1) % NSLOT].wait()
                nh_x = issue_x(c + 1)
            h_g.wait()
            h_x.wait()

            rows_c = rows[s]
            xb_c = xb[s]

            @plsc.parallel_loop(0, CH, 1)
            def add_body(ir):
                for t in range(D // LANES):
                    sl = pl.ds(t * LANES, LANES)
                    plsc.addupdate(xb_c.at[ir, sl], rows_c[ir, sl])

            h_out[s] = pltpu.async_copy(
                xb_c, out_hbm.at[pl.ds(r_base + c * CH, CH)], osem[s]
            )
            h_g, h_x = nh_g, nh_x

        for h in h_out:
            if h is not None:
                h.wait()

    return k(x2, yflat, w)


def kernel(x, y, W_lookup):
    b, h, d = x.shape
    x2 = jnp.transpose(x, (1, 0, 2)).reshape(b * h, d)
    yflat = jnp.transpose(y).reshape(b * h).astype(jnp.int32)
    out2 = _lookup_add(x2, yflat, W_lookup)
    return jnp.transpose(out2.reshape(h, b, d), (1, 0, 2))


# speedup vs baseline: 6.9548x; 1.0452x over previous
import functools

import jax
import jax.numpy as jnp
from jax import lax
from jax.experimental import pallas as pl
from jax.experimental.pallas import tpu as pltpu
from jax.experimental.pallas import tpu_sc as plsc

D = 128
LANES = 16
NC = 2
NS = 16
NW = NC * NS
CH = 128
NSLOT = 3


def _lookup_add(x2, yflat, w):
    n = x2.shape[0]
    n_w = n // NW
    n_chunks = n_w // CH
    mesh = plsc.VectorSubcoreMesh(core_axis_name="c", subcore_axis_name="s")

    idx_template = (
        jnp.arange(NSLOT * CH, dtype=jnp.int32).reshape(NSLOT, CH)
    )

    scratch = (
        [pltpu.VMEM((n_w,), jnp.int32)]
        + [pltpu.VMEM((NSLOT, CH), jnp.int32)]
        + [pltpu.VMEM((NSLOT, CH), jnp.int32)]
        + [pltpu.VMEM((CH, D), jnp.float32) for _ in range(NSLOT)]
        + [pltpu.VMEM_SHARED((NS * NSLOT * CH, D), jnp.float32)]
        + [pltpu.SemaphoreType.DMA for _ in range(4 * NSLOT)]
    )

    @functools.partial(
        pl.kernel,
        mesh=mesh,
        out_type=jax.ShapeDtypeStruct(x2.shape, jnp.float32),
        scratch_types=scratch,
    )
    def k(x_hbm, y_hbm, w_hbm, idxt_hbm, out_hbm, idx_all, idxt, idxa,
          *bufs_and_sems):
        rows = bufs_and_sems[:NSLOT]
        shared = bufs_and_sems[NSLOT]
        gsem = bufs_and_sems[NSLOT + 1:2 * NSLOT + 1]
        xsem = bufs_and_sems[2 * NSLOT + 1:3 * NSLOT + 1]
        asem = bufs_and_sems[3 * NSLOT + 1:4 * NSLOT + 1]
        osem = bufs_and_sems[4 * NSLOT + 1:]

        sid = lax.axis_index("s")
        wid = sid * NC + lax.axis_index("c")
        r_base = wid * n_w
        s_base = sid * (NSLOT * CH)

        pltpu.sync_copy(y_hbm.at[pl.ds(r_base, n_w)], idx_all)
        pltpu.sync_copy(idxt_hbm, idxt)
        for sl in range(NSLOT):
            for t in range(CH // LANES):
                dl = pl.ds(t * LANES, LANES)
                idxa[sl, dl] = idxt[sl, dl] + s_base

        def issue_gather(c):
            s = c % NSLOT
            return pltpu.async_copy(
                w_hbm.at[idx_all.at[pl.ds(c * CH, CH)]],
                rows[s],
                gsem[s],
            )

        def issue_x(c):
            s = c % NSLOT
            return pltpu.async_copy(
                x_hbm.at[pl.ds(r_base + c * CH, CH)],
                shared.at[pl.ds(s_base + s * CH, CH)],
                xsem[s],
            )

        def issue_add(c):
            s = c % NSLOT
            return pltpu.async_copy(
                rows[s], shared.at[idxa.at[s]], asem[s], add=True
            )

        def issue_out(c):
            s = c % NSLOT
            return pltpu.async_copy(
                shared.at[pl.ds(s_base + s * CH, CH)],
                out_hbm.at[pl.ds(r_base + c * CH, CH)],
                osem[s],
            )

        h_g = [None] * NSLOT
        h_x = [None] * NSLOT
        h_add = [None] * NSLOT
        h_out = [None] * NSLOT
        for c in range(min(NSLOT, n_chunks)):
            h_g[c] = issue_gather(c)
            h_x[c] = issue_x(c)

        for c in range(n_chunks):
            s = c % NSLOT
            h_g[s].wait()
            h_x[s].wait()
            h_add[s] = issue_add(c)
            if c >= 1:
                sp = (c - 1) % NSLOT
                h_add[sp].wait()
                h_add[sp] = None
                h_out[sp] = issue_out(c - 1)
                if c - 1 + NSLOT < n_chunks:
                    h_g[sp] = issue_gather(c - 1 + NSLOT)
            if c >= 2:
                so = (c - 2) % NSLOT
                h_out[so].wait()
                h_out[so] = None
                if c - 2 + NSLOT < n_chunks:
                    h_x[so] = issue_x(c - 2 + NSLOT)

        for c in (n_chunks - 1,):
            s = c % NSLOT
            if h_add[s] is not None:
                h_add[s].wait()
                h_out[s] = issue_out(c)
        for h in h_out:
            if h is not None:
                h.wait()

    return k(x2, yflat, w, idx_template)


def kernel(x, y, W_lookup):
    b, h, d = x.shape
    x2 = jnp.transpose(x, (1, 0, 2)).reshape(b * h, d)
    yflat = jnp.transpose(y).reshape(b * h).astype(jnp.int32)
    out2 = _lookup_add(x2, yflat, W_lookup)
    return jnp.transpose(out2.reshape(h, b, d), (1, 0, 2))


# speedup vs baseline: 7.0936x; 1.0200x over previous
import functools

import jax
import jax.numpy as jnp
from jax import lax
from jax.experimental import pallas as pl
from jax.experimental.pallas import tpu as pltpu
from jax.experimental.pallas import tpu_sc as plsc

D = 128
LANES = 16
NC = 2
NS = 16
NW = NC * NS
CH = 128
NSLOT = 3


def _lookup_add(x2, y2, w):
    n, _ = x2.shape
    hist, bsz = y2.shape
    n_chunks = hist
    b_w = bsz // NW
    mesh = plsc.VectorSubcoreMesh(core_axis_name="c", subcore_axis_name="s")

    idx_template = (
        jnp.arange(NSLOT * CH, dtype=jnp.int32).reshape(NSLOT, CH)
    )

    scratch = (
        [pltpu.VMEM((hist, b_w), jnp.int32)]
        + [pltpu.VMEM((NSLOT, CH), jnp.int32)]
        + [pltpu.VMEM((NSLOT, CH), jnp.int32)]
        + [pltpu.VMEM((CH, D), jnp.float32) for _ in range(NSLOT)]
        + [pltpu.VMEM_SHARED((NS * NSLOT * CH, D), jnp.float32)]
        + [pltpu.SemaphoreType.DMA for _ in range(4 * NSLOT)]
    )

    @functools.partial(
        pl.kernel,
        mesh=mesh,
        out_type=jax.ShapeDtypeStruct(x2.shape, jnp.float32),
        scratch_types=scratch,
    )
    def k(x_hbm, y_hbm, w_hbm, idxt_hbm, out_hbm, idx_all, idxt, idxa,
          *bufs_and_sems):
        rows = bufs_and_sems[:NSLOT]
        shared = bufs_and_sems[NSLOT]
        gsem = bufs_and_sems[NSLOT + 1:2 * NSLOT + 1]
        xsem = bufs_and_sems[2 * NSLOT + 1:3 * NSLOT + 1]
        asem = bufs_and_sems[3 * NSLOT + 1:4 * NSLOT + 1]
        osem = bufs_and_sems[4 * NSLOT + 1:]

        sid = lax.axis_index("s")
        wid = sid * NC + lax.axis_index("c")
        b_base = wid * b_w
        s_base = sid * (NSLOT * CH)

        pltpu.sync_copy(y_hbm.at[:, pl.ds(b_base, b_w)], idx_all)
        pltpu.sync_copy(idxt_hbm, idxt)
        for sl in range(NSLOT):
            for t in range(CH // LANES):
                dl = pl.ds(t * LANES, LANES)
                idxa[sl, dl] = idxt[sl, dl] + s_base

        def issue_gather(c):
            s = c % NSLOT
            return pltpu.async_copy(
                w_hbm.at[idx_all.at[c]],
                rows[s],
                gsem[s],
            )

        def issue_x(c):
            s = c % NSLOT
            return pltpu.async_copy(
                x_hbm.at[pl.ds(c * bsz + b_base, CH)],
                shared.at[pl.ds(s_base + s * CH, CH)],
                xsem[s],
            )

        def issue_add(c):
            s = c % NSLOT
            return pltpu.async_copy(
                rows[s], shared.at[idxa.at[s]], asem[s], add=True
            )

        def issue_out(c):
            s = c % NSLOT
            return pltpu.async_copy(
                shared.at[pl.ds(s_base + s * CH, CH)],
                out_hbm.at[pl.ds(c * bsz + b_base, CH)],
                osem[s],
            )

        h_g = [None] * NSLOT
        h_x = [None] * NSLOT
        h_add = [None] * NSLOT
        h_out = [None] * NSLOT
        for c in range(min(NSLOT, n_chunks)):
            h_g[c] = issue_gather(c)
            h_x[c] = issue_x(c)

        for c in range(n_chunks):
            s = c % NSLOT
            h_g[s].wait()
            h_x[s].wait()
            h_add[s] = issue_add(c)
            if c >= 1:
                sp = (c - 1) % NSLOT
                h_add[sp].wait()
                h_add[sp] = None
                h_out[sp] = issue_out(c - 1)
                if c - 1 + NSLOT < n_chunks:
                    h_g[sp] = issue_gather(c - 1 + NSLOT)
            if c >= 2:
                so = (c - 2) % NSLOT
                h_out[so].wait()
                h_out[so] = None
                if c - 2 + NSLOT < n_chunks:
                    h_x[so] = issue_x(c - 2 + NSLOT)

        for c in (n_chunks - 1,):
            s = c % NSLOT
            if h_add[s] is not None:
                h_add[s].wait()
                h_out[s] = issue_out(c)
        for h in h_out:
            if h is not None:
                h.wait()

    return k(x2, y2, w, idx_template)


def kernel(x, y, W_lookup):
    b, h, d = x.shape
    x2 = jnp.transpose(x, (1, 0, 2)).reshape(b * h, d)
    y2 = jnp.transpose(y).astype(jnp.int32)
    out2 = _lookup_add(x2, y2, W_lookup)
    return jnp.transpose(out2.reshape(h, b, d), (1, 0, 2))
